# Initial kernel scaffold; baseline (speedup 1.0000x reference)
import jax
import jax.numpy as jnp
from jax.experimental import pallas as pl


def kernel(tuples, tables):
    raise NotImplementedError("write your pallas kernel here")


# speedup vs baseline: 1.0842x; 1.0842x over previous
import functools

import jax
import jax.numpy as jnp
from jax import lax
from jax.experimental import pallas as pl
from jax.experimental.pallas import tpu as pltpu
from jax.experimental.pallas import tpu_sc as plsc

A = 26
V = 100000
D = 64
B = 16384
TOTAL = B * A

NC, NS = 2, 16
NW = NC * NS
ROWS_W = TOTAL // NW
IDXW = 128
CHUNK = 512
NJ = CHUNK // IDXW
NCHUNK = ROWS_W // CHUNK
CH128_W = ROWS_W // IDXW


def _body(idx_hbm, tab_hbm, out_hbm, idx_v, rows_v, gsem):
    wid = lax.axis_index("s") * NC + lax.axis_index("c")

    def step(c, _):
        base = wid * CH128_W + c * NJ
        pltpu.sync_copy(idx_hbm.at[pl.ds(base, NJ)], idx_v.at[0])
        for j in range(NJ):
            pltpu.async_copy(tab_hbm.at[idx_v.at[0, j]], rows_v.at[0, j], gsem)
        for j in range(NJ):
            pltpu.make_async_copy(
                tab_hbm.at[idx_v.at[0, j]], rows_v.at[0, j], gsem
            ).wait()
        pltpu.sync_copy(rows_v.at[0], out_hbm.at[pl.ds(base, NJ)])
        return ()

    lax.fori_loop(0, NCHUNK, step, ())


@functools.partial(jax.jit, static_argnames=())
def _gather(flat_idx, flat_tab):
    mesh = plsc.VectorSubcoreMesh(core_axis_name="c", subcore_axis_name="s")
    f = pl.kernel(
        _body,
        out_type=jax.ShapeDtypeStruct((TOTAL // IDXW, IDXW, D), jnp.float32),
        mesh=mesh,
        scratch_types=[
            pltpu.VMEM((1, NJ, IDXW), jnp.int32),
            pltpu.VMEM((1, NJ, IDXW, D), jnp.float32),
            pltpu.SemaphoreType.DMA,
        ],
        compiler_params=pltpu.CompilerParams(use_tc_tiling_on_sc=False),
    )
    return f(flat_idx, flat_tab)


def kernel(tuples, tables):
    offs = (jnp.arange(A, dtype=jnp.int32) * V)[None, :]
    flat_idx = (tuples + offs).reshape(TOTAL // IDXW, IDXW)
    flat_tab = tables.reshape(A * V, D)
    out = _gather(flat_idx, flat_tab)
    return out.reshape(B, A * D)
